# baseline (speedup 1.0000x reference)
import functools

import jax
import jax.numpy as jnp
from jax import lax
from jax.experimental import pallas as pl
from jax.experimental.pallas import tpu as pltpu
from jax.experimental.pallas import tpu_sc as plsc

V = 1000
D = 128
N_OUT = 100000

NC = 2
NS = 16
NW = NC * NS

CHUNK = 128
FULL_CH = 24
SB = 1
NBUF = 6
NSUPER = FULL_CH // SB
SROWS = SB * CHUNK

SIZE_A = FULL_CH * CHUNK + 56
SIZE_B = FULL_CH * CHUNK + 32
N_A = 28
TAIL_A = 56
TAIL_B = 32


def _table_kernel(emb_ref, w_ref, b_ref, out_ref):
    emb = emb_ref[...]
    row_ids = lax.broadcasted_iota(jnp.int32, emb.shape, 0)
    emb = jnp.where(row_ids == 0, 0.0, emb)
    emb = jnp.maximum(emb, 0.0)
    acc = jnp.dot(emb, w_ref[...], preferred_element_type=jnp.float32)
    out_ref[...] = jnp.maximum(acc + b_ref[...], 0.0)


def _build_table(emb_table, W, b):
    return pl.pallas_call(
        _table_kernel,
        out_shape=jax.ShapeDtypeStruct((V, D), jnp.float32),
    )(emb_table, W, b.reshape(1, D))


_sc_mesh = plsc.VectorSubcoreMesh(core_axis_name="c", subcore_axis_name="s")


@functools.partial(
    pl.kernel,
    mesh=_sc_mesh,
    out_type=jax.ShapeDtypeStruct((N_OUT, D), jnp.float32),
    scratch_types=[
        pltpu.VMEM((SIZE_A,), jnp.int32),
        pltpu.VMEM_SHARED((V, D), jnp.float32),
        [pltpu.VMEM((SROWS, D), jnp.float32) for _ in range(NBUF)],
        pltpu.VMEM((TAIL_A, D), jnp.float32),
        [pltpu.SemaphoreType.DMA for _ in range(NBUF)],
        [pltpu.SemaphoreType.DMA for _ in range(NBUF)],
        [pltpu.SemaphoreType.DMA for _ in range(2)],
    ],
)
def _gather_kernel(
    idx_hbm, table_hbm, out_hbm, idx_v, table_sp, rows, tail_v, gsems, ssems,
    tsems,
):
    sid = lax.axis_index("s")
    wid = sid * NC + lax.axis_index("c")
    is_a = wid < N_A
    base = jnp.where(is_a, wid * SIZE_A, N_A * SIZE_A + (wid - N_A) * SIZE_B)

    @pl.when(sid == 0)
    def _():
        pltpu.make_async_copy(table_hbm, table_sp, tsems[0]).start()

    @pl.when(is_a)
    def _():
        pltpu.sync_copy(idx_hbm.at[pl.ds(base, SIZE_A)], idx_v)

    @pl.when(jnp.logical_not(is_a))
    def _():
        pltpu.sync_copy(
            idx_hbm.at[pl.ds(base, SIZE_B)], idx_v.at[pl.ds(0, SIZE_B)]
        )

    @pl.when(sid == 0)
    def _():
        pltpu.make_async_copy(table_hbm, table_sp, tsems[0]).wait()

    plsc.subcore_barrier()

    tail_off = FULL_CH * CHUNK

    def tail_gather_desc(n):
        return pltpu.make_async_copy(
            table_sp.at[idx_v.at[pl.ds(tail_off, n)]],
            tail_v.at[pl.ds(0, n)],
            tsems[0],
        )

    def tail_scatter_desc(n):
        return pltpu.make_async_copy(
            tail_v.at[pl.ds(0, n)],
            out_hbm.at[pl.ds(base + tail_off, n)],
            tsems[1],
        )

    @pl.when(is_a)
    def _():
        tail_gather_desc(TAIL_A).start()

    @pl.when(jnp.logical_not(is_a))
    def _():
        tail_gather_desc(TAIL_B).start()

    def fire_gathers(g, b):
        gathers = []
        for k in range(SB):
            c = g * SB + k
            gathers.append(
                pltpu.async_copy(
                    table_sp.at[idx_v.at[pl.ds(c * CHUNK, CHUNK)]],
                    rows[b].at[pl.ds(k * CHUNK, CHUNK)],
                    gsems[b],
                )
            )
        return gathers

    def scatter_desc(g, b):
        return pltpu.make_async_copy(
            rows[b], out_hbm.at[pl.ds(base + g * SROWS, SROWS)], ssems[b]
        )

    def fire_scatter(g, b, gathers):
        for gd in gathers:
            gd.wait()
        scatter_desc(g, b).start()

    for b in range(NBUF):
        fire_scatter(b, b, fire_gathers(b, b))

    def outer(o, carry):
        for b in range(NBUF):
            g = NBUF + o * NBUF + b
            scatter_desc(g - NBUF, b).wait()
            fire_scatter(g, b, fire_gathers(g, b))
        return carry

    lax.fori_loop(0, (NSUPER - NBUF) // NBUF, outer, 0)

    @pl.when(is_a)
    def _():
        tail_gather_desc(TAIL_A).wait()
        tail_scatter_desc(TAIL_A).start()

    @pl.when(jnp.logical_not(is_a))
    def _():
        tail_gather_desc(TAIL_B).wait()
        tail_scatter_desc(TAIL_B).start()

    for b in range(NBUF):
        scatter_desc(NSUPER - NBUF + b, b).wait()

    @pl.when(is_a)
    def _():
        tail_scatter_desc(TAIL_A).wait()

    @pl.when(jnp.logical_not(is_a))
    def _():
        tail_scatter_desc(TAIL_B).wait()


def kernel(z, emb_table, W, b):
    table = _build_table(emb_table, W, b)
    return _gather_kernel(z.astype(jnp.int32), table)


# speedup vs baseline: 1.0101x; 1.0101x over previous
import functools

import jax
import jax.numpy as jnp
from jax import lax
from jax.experimental import pallas as pl
from jax.experimental.pallas import tpu as pltpu
from jax.experimental.pallas import tpu_sc as plsc

V = 1000
D = 128
N_OUT = 100000

NC = 2
NS = 16
NW = NC * NS

CHUNK = 128
FULL_CH = 24
SB = 2
NBUF = 3
NSUPER = FULL_CH // SB
SROWS = SB * CHUNK

SIZE_A = FULL_CH * CHUNK + 56
SIZE_B = FULL_CH * CHUNK + 32
N_A = 28
TAIL_A = 56
TAIL_B = 32


def _table_kernel(emb_ref, w_ref, b_ref, out_ref):
    emb = emb_ref[...]
    row_ids = lax.broadcasted_iota(jnp.int32, emb.shape, 0)
    emb = jnp.where(row_ids == 0, 0.0, emb)
    emb = jnp.maximum(emb, 0.0)
    acc = jnp.dot(emb, w_ref[...], preferred_element_type=jnp.float32)
    out_ref[...] = jnp.maximum(acc + b_ref[...], 0.0)


def _build_table(emb_table, W, b):
    return pl.pallas_call(
        _table_kernel,
        out_shape=jax.ShapeDtypeStruct((V, D), jnp.float32),
    )(emb_table, W, b.reshape(1, D))


_sc_mesh = plsc.VectorSubcoreMesh(core_axis_name="c", subcore_axis_name="s")


@functools.partial(
    pl.kernel,
    mesh=_sc_mesh,
    out_type=jax.ShapeDtypeStruct((N_OUT, D), jnp.float32),
    scratch_types=[
        pltpu.VMEM((SIZE_A,), jnp.int32),
        pltpu.VMEM_SHARED((V, D), jnp.float32),
        [pltpu.VMEM((SROWS, D), jnp.float32) for _ in range(NBUF)],
        pltpu.VMEM((TAIL_A, D), jnp.float32),
        [pltpu.SemaphoreType.DMA for _ in range(NBUF)],
        [pltpu.SemaphoreType.DMA for _ in range(NBUF)],
        [pltpu.SemaphoreType.DMA for _ in range(2)],
    ],
)
def _gather_kernel(
    idx_hbm, table_hbm, out_hbm, idx_v, table_sp, rows, tail_v, gsems, ssems,
    tsems,
):
    sid = lax.axis_index("s")
    wid = sid * NC + lax.axis_index("c")
    is_a = wid < N_A
    base = jnp.where(is_a, wid * SIZE_A, N_A * SIZE_A + (wid - N_A) * SIZE_B)

    @pl.when(sid == 0)
    def _():
        pltpu.make_async_copy(table_hbm, table_sp, tsems[0]).start()

    @pl.when(is_a)
    def _():
        pltpu.sync_copy(idx_hbm.at[pl.ds(base, SIZE_A)], idx_v)

    @pl.when(jnp.logical_not(is_a))
    def _():
        pltpu.sync_copy(
            idx_hbm.at[pl.ds(base, SIZE_B)], idx_v.at[pl.ds(0, SIZE_B)]
        )

    @pl.when(sid == 0)
    def _():
        pltpu.make_async_copy(table_hbm, table_sp, tsems[0]).wait()

    plsc.subcore_barrier()

    tail_off = FULL_CH * CHUNK

    def tail_gather_desc(n):
        return pltpu.make_async_copy(
            table_sp.at[idx_v.at[pl.ds(tail_off, n)]],
            tail_v.at[pl.ds(0, n)],
            tsems[0],
        )

    def tail_scatter_desc(n):
        return pltpu.make_async_copy(
            tail_v.at[pl.ds(0, n)],
            out_hbm.at[pl.ds(base + tail_off, n)],
            tsems[1],
        )

    @pl.when(is_a)
    def _():
        tail_gather_desc(TAIL_A).start()

    @pl.when(jnp.logical_not(is_a))
    def _():
        tail_gather_desc(TAIL_B).start()

    def fire_gathers(g, b):
        gathers = []
        for k in range(SB):
            c = g * SB + k
            gathers.append(
                pltpu.async_copy(
                    table_sp.at[idx_v.at[pl.ds(c * CHUNK, CHUNK)]],
                    rows[b].at[pl.ds(k * CHUNK, CHUNK)],
                    gsems[b],
                )
            )
        return gathers

    def scatter_desc(g, b):
        return pltpu.make_async_copy(
            rows[b], out_hbm.at[pl.ds(base + g * SROWS, SROWS)], ssems[b]
        )

    def fire_scatter(g, b, gathers):
        for gd in gathers:
            gd.wait()
        scatter_desc(g, b).start()

    for b in range(NBUF):
        fire_scatter(b, b, fire_gathers(b, b))

    def outer(o, carry):
        for b in range(NBUF):
            g = NBUF + o * NBUF + b
            scatter_desc(g - NBUF, b).wait()
            fire_scatter(g, b, fire_gathers(g, b))
        return carry

    lax.fori_loop(0, (NSUPER - NBUF) // NBUF, outer, 0)

    @pl.when(is_a)
    def _():
        tail_gather_desc(TAIL_A).wait()
        tail_scatter_desc(TAIL_A).start()

    @pl.when(jnp.logical_not(is_a))
    def _():
        tail_gather_desc(TAIL_B).wait()
        tail_scatter_desc(TAIL_B).start()

    for b in range(NBUF):
        scatter_desc(NSUPER - NBUF + b, b).wait()

    @pl.when(is_a)
    def _():
        tail_scatter_desc(TAIL_A).wait()

    @pl.when(jnp.logical_not(is_a))
    def _():
        tail_scatter_desc(TAIL_B).wait()


def kernel(z, emb_table, W, b):
    table = _build_table(emb_table, W, b)
    return _gather_kernel(z.astype(jnp.int32), table)
